# baseline (speedup 1.0000x reference)
import functools

import jax
import jax.numpy as jnp
from jax import lax
from jax.experimental import pallas as pl
from jax.experimental.pallas import tpu as pltpu
from jax.experimental.pallas import tpu_sc as plsc

N = 10000
E = 160000
D = 256
EPS = 1e-5


_NB = 1000


def _node_mm_body(h_ref, w_ref, b_ref, out_ref):
    out_ref[0] = (
        jnp.dot(h_ref[...], w_ref[0], preferred_element_type=jnp.float32)
        + b_ref[0]
    )


def _node_mm(h, Wstack, bstack):
    grid = (N // _NB, 5)
    return pl.pallas_call(
        _node_mm_body,
        grid=grid,
        in_specs=[
            pl.BlockSpec((_NB, D), lambda i, j: (i, 0)),
            pl.BlockSpec((1, D, D), lambda i, j: (j, 0, 0)),
            pl.BlockSpec((1, 1, D), lambda i, j: (j, 0, 0)),
        ],
        out_specs=pl.BlockSpec((1, _NB, D), lambda i, j: (j, i, 0)),
        out_shape=jax.ShapeDtypeStruct((5, N, D), jnp.float32),
    )(h, Wstack, bstack)


_GW = 32
_EPT = E // _GW
_GB = 40
_GNB = _EPT // _GB


def _sc_gather_body(src3, dst3, w2h_hbm, w3h_hbm,
                    g_out,
                    srcb, dstb, b2a, b2b, b3a, b3b,
                    sidx, sg0, sg1, sw0, sw1):
    wid = lax.axis_index("s") * 2 + lax.axis_index("c")
    base = wid * _EPT
    b2 = [b2a, b2b]
    b3 = [b3a, b3b]
    sg = [sg0, sg1]
    sw = [sw0, sw1]

    pltpu.async_copy(src3.at[wid], srcb, sidx).wait()
    pltpu.async_copy(dst3.at[wid], dstb, sidx).wait()

    def fire_g(b, j):
        pltpu.async_copy(w2h_hbm.at[srcb.at[b]], b2[j], sg[j])
        pltpu.async_copy(w3h_hbm.at[dstb.at[b]], b3[j], sg[j])

    def wait_g(b, j):
        pltpu.make_async_copy(w2h_hbm.at[srcb.at[b]], b2[j], sg[j]).wait()
        pltpu.make_async_copy(w3h_hbm.at[dstb.at[b]], b3[j], sg[j]).wait()

    def fire_w(b, j):
        off = base + b * _GB
        pltpu.async_copy(b2[j], g_out.at[pl.ds(off, _GB)], sw[j])

    def wait_w(b, j):
        off = base + b * _GB
        pltpu.make_async_copy(b2[j], g_out.at[pl.ds(off, _GB)], sw[j]).wait()

    def compute(j):
        def row_body(r, carry):
            for cv in range(16):
                sl = pl.ds(cv * 16, 16)
                b2[j][r, sl] = b2[j][r, sl] + b3[j][r, sl]
            return carry
        lax.fori_loop(0, _GB, row_body, 0)

    fire_g(0, 0)

    def outer(r, carry):
        for jj in range(2):
            b = r * 2 + jj

            @pl.when(b < _GNB)
            def _():
                @pl.when(b >= 1)
                def _():
                    wait_w(b - 1, 1 - jj)

                @pl.when(b + 1 < _GNB)
                def _():
                    fire_g(b + 1, 1 - jj)

                wait_g(b, jj)
                compute(jj)
                fire_w(b, jj)
        return carry

    lax.fori_loop(0, (_GNB + 2) // 2, outer, 0)
    wait_w(_GNB - 1, (_GNB - 1) % 2)


def _sc_gather(src3, dst3, w2h, w3h):
    mesh = plsc.VectorSubcoreMesh(core_axis_name="c", subcore_axis_name="s")
    f = functools.partial(
        pl.kernel,
        out_type=jax.ShapeDtypeStruct((E, D), jnp.float32),
        mesh=mesh,
        scratch_types=[
            pltpu.VMEM((_GNB, _GB), jnp.int32),
            pltpu.VMEM((_GNB, _GB), jnp.int32),
            pltpu.VMEM((_GB, D), jnp.float32),
            pltpu.VMEM((_GB, D), jnp.float32),
            pltpu.VMEM((_GB, D), jnp.float32),
            pltpu.VMEM((_GB, D), jnp.float32),
            pltpu.SemaphoreType.DMA,
            pltpu.SemaphoreType.DMA,
            pltpu.SemaphoreType.DMA,
            pltpu.SemaphoreType.DMA,
            pltpu.SemaphoreType.DMA,
        ],
    )(_sc_gather_body)
    return f(src3, dst3, w2h, w3h)


_EB = 2000


def _edge1_body(e_ref, g_ref, w1_ref, b1_ref, stats_ref):
    i = pl.program_id(0)
    w1e = jnp.dot(e_ref[...], w1_ref[...], preferred_element_type=jnp.float32)
    pre = w1e + b1_ref[...] + g_ref[...]

    @pl.when(i == 0)
    def _():
        stats_ref[...] = jnp.zeros_like(stats_ref)

    stats_ref[0:1, :] += jnp.sum(pre, axis=0, keepdims=True)
    stats_ref[1:2, :] += jnp.sum(pre * pre, axis=0, keepdims=True)


def _edge1(e, g, W1, b1):
    grid = (E // _EB,)
    return pl.pallas_call(
        _edge1_body,
        grid=grid,
        in_specs=[
            pl.BlockSpec((_EB, 16), lambda i: (i, 0)),
            pl.BlockSpec((_EB, D), lambda i: (i, 0)),
            pl.BlockSpec((16, D), lambda i: (0, 0)),
            pl.BlockSpec((1, D), lambda i: (0, 0)),
        ],
        out_specs=pl.BlockSpec((8, D), lambda i: (0, 0)),
        out_shape=jax.ShapeDtypeStruct((8, D), jnp.float32),
    )(e, g, W1, b1)


def _enew_block(e_blk, g_blk, stats, b1, ge, be, w1, wemb):
    mean = stats[0:1, :] * (1.0 / E)
    var = stats[1:2, :] * (1.0 / E) - mean * mean
    rstd = lax.rsqrt(var + EPS)
    w1e = jnp.dot(e_blk, w1, preferred_element_type=jnp.float32)
    pre = w1e + b1 + g_blk
    norm = (pre - mean) * (rstd * ge) + be
    e0 = jnp.dot(e_blk, wemb, preferred_element_type=jnp.float32)
    return e0 + jnp.maximum(norm, 0.0)


def _edge2a_body(e_ref, g_ref, stats_ref, b1_ref, ge_ref, be_ref,
                 w1_ref, wemb_ref, weta_ref, pay_ref):
    e_new = _enew_block(e_ref[...], g_ref[...], stats_ref[...], b1_ref[...],
                        ge_ref[...], be_ref[...], w1_ref[...], wemb_ref[...])
    sig = jax.nn.sigmoid(e_new)
    sigma = jnp.dot(sig, weta_ref[...], preferred_element_type=jnp.float32)
    pay_ref[0] = sigma[:, 0:128]
    pay_ref[1] = sigma[:, 128:256]


def _edge2a(e, g, stats, b1, gamma_e, beta_e, W1, W_emb_e, W_eta):
    grid = (E // _EB,)
    return pl.pallas_call(
        _edge2a_body,
        grid=grid,
        in_specs=[
            pl.BlockSpec((_EB, 16), lambda i: (i, 0)),
            pl.BlockSpec((_EB, D), lambda i: (i, 0)),
            pl.BlockSpec((8, D), lambda i: (0, 0)),
            pl.BlockSpec((1, D), lambda i: (0, 0)),
            pl.BlockSpec((1, D), lambda i: (0, 0)),
            pl.BlockSpec((1, D), lambda i: (0, 0)),
            pl.BlockSpec((16, D), lambda i: (0, 0)),
            pl.BlockSpec((16, D), lambda i: (0, 0)),
            pl.BlockSpec((D, D), lambda i: (0, 0)),
        ],
        out_specs=pl.BlockSpec((2, _EB, _CC), lambda i: (0, i, 0)),
        out_shape=jax.ShapeDtypeStruct((2, E, _CC), jnp.float32),
    )(e, g, stats, b1, gamma_e, beta_e, W1, W_emb_e, W_eta)


def _edge2b_body(g_ref, e_ref, stats_ref, b1_ref, ge_ref, be_ref,
                 w1_ref, wemb_ref, enew_ref):
    enew_ref[...] = _enew_block(
        e_ref[...], g_ref[...], stats_ref[...], b1_ref[...],
        ge_ref[...], be_ref[...], w1_ref[...], wemb_ref[...])


def _edge2b(g, e, stats, b1, gamma_e, beta_e, W1, W_emb_e):
    grid = (E // _EB,)
    return pl.pallas_call(
        _edge2b_body,
        grid=grid,
        in_specs=[
            pl.BlockSpec((_EB, D), lambda i: (i, 0)),
            pl.BlockSpec((_EB, 16), lambda i: (i, 0)),
            pl.BlockSpec((8, D), lambda i: (0, 0)),
            pl.BlockSpec((1, D), lambda i: (0, 0)),
            pl.BlockSpec((1, D), lambda i: (0, 0)),
            pl.BlockSpec((1, D), lambda i: (0, 0)),
            pl.BlockSpec((16, D), lambda i: (0, 0)),
            pl.BlockSpec((16, D), lambda i: (0, 0)),
        ],
        out_specs=pl.BlockSpec((_EB, D), lambda i: (i, 0)),
        out_shape=jax.ShapeDtypeStruct((E, D), jnp.float32),
        input_output_aliases={0: 0},
    )(g, e, stats, b1, gamma_e, beta_e, W1, W_emb_e)


_SB = 80
_EPS_T = E // 16
_SNB = _EPS_T // _SB
_NPAD = 10112
_RPT = _NPAD // 16
_CC = 128


def _sc_seg_body(src3t, dst3, sig_hbm, vh_hbm, zeros_hbm, out_hbm,
                 isa, isb, ida, idb, i2a, i2b, isca, iscb, sa, sb, va, vb,
                 acc, si0, si1, sv0, sv1, sg0, sg1, ss0, ss1):
    core = lax.axis_index("c")
    tile = lax.axis_index("s")
    rbase = tile * _RPT
    cN = core * N
    ibs = [isa, isb]
    ibd = [ida, idb]
    i2 = [i2a, i2b]
    isc = [isca, iscb]
    sgb = [sa, sb]
    vhb = [va, vb]
    si = [si0, si1]
    sv = [sv0, sv1]
    sg = [sg0, sg1]
    ss = [ss0, ss1]

    for phase in range(2):
        pltpu.sync_copy(zeros_hbm, acc.at[pl.ds(rbase, _RPT)])
        plsc.subcore_barrier()

        def fire_idx(b, j):
            pltpu.async_copy(dst3.at[tile, b], ibd[j], si[j])
            if phase == 0:
                pltpu.async_copy(src3t.at[tile, b], ibs[j], si[j])

        def wait_idx(b, j):
            pltpu.make_async_copy(dst3.at[tile, b], ibd[j], si[j]).wait()
            if phase == 0:
                pltpu.make_async_copy(src3t.at[tile, b], ibs[j],
                                      si[j]).wait()

        def build(j):
            for q in range(_SB // 16):
                qs = pl.ds(q * 16, 16)
                isc[j][qs] = ibd[j][qs]
                if phase == 0:
                    i2[j][qs] = ibs[j][qs] + cN

        def fire_sig(b, j):
            off = tile * _EPS_T + b * _SB
            pltpu.async_copy(sig_hbm.at[core, pl.ds(off, _SB)],
                             sgb[j], sg[j])

        def wait_sig(b, j):
            off = tile * _EPS_T + b * _SB
            pltpu.make_async_copy(sig_hbm.at[core, pl.ds(off, _SB)],
                                  sgb[j], sg[j]).wait()

        def fire_vh(j):
            pltpu.async_copy(vh_hbm.at[i2[j]], vhb[j], sv[j])

        def wait_vh(j):
            pltpu.make_async_copy(vh_hbm.at[i2[j]], vhb[j], sv[j]).wait()

        def compute(j):
            def row_body(r, carry):
                for cv in range(_CC // 16):
                    slc = pl.ds(cv * 16, 16)
                    vhb[j][r, slc] = vhb[j][r, slc] * sgb[j][r, slc]
                return carry
            lax.fori_loop(0, _SB, row_body, 0)

        def fire_scat(j):
            buf = vhb[j] if phase == 0 else sgb[j]
            pltpu.async_copy(buf, acc.at[isc[j]], ss[j], add=True)

        def wait_scat(j):
            buf = vhb[j] if phase == 0 else sgb[j]
            pltpu.make_async_copy(buf, acc.at[isc[j]], ss[j]).wait()

        fire_idx(0, 0)
        fire_idx(1, 1)
        wait_idx(0, 0)
        build(0)
        if phase == 0:
            fire_vh(0)
        fire_sig(0, 0)

        def outer(r, carry):
            for jj in range(2):
                b = r * 2 + jj

                @pl.when(b < _SNB)
                def _():
                    @pl.when(b >= 1)
                    def _():
                        wait_scat(1 - jj)

                    @pl.when(b + 1 < _SNB)
                    def _():
                        wait_idx(b + 1, 1 - jj)
                        build(1 - jj)
                        if phase == 0:
                            fire_vh(1 - jj)
                        fire_sig(b + 1, 1 - jj)

                    @pl.when(b + 2 < _SNB)
                    def _():
                        fire_idx(b + 2, jj)

                    wait_sig(b, jj)
                    if phase == 0:
                        wait_vh(jj)
                        compute(jj)
                    fire_scat(jj)
            return carry

        lax.fori_loop(0, (_SNB + 2) // 2, outer, 0)
        wait_scat((_SNB - 1) % 2)
        plsc.subcore_barrier()
        pltpu.sync_copy(acc.at[pl.ds(rbase, _RPT)],
                        out_hbm.at[core + 2 * phase, pl.ds(rbase, _RPT)])


def _sc_seg(src3t, dst3, sigma2, vhflat, zeros_nc):
    mesh = plsc.VectorSubcoreMesh(core_axis_name="c", subcore_axis_name="s")
    f = functools.partial(
        pl.kernel,
        out_type=jax.ShapeDtypeStruct((4, _NPAD, _CC), jnp.float32),
        mesh=mesh,
        scratch_types=[
            pltpu.VMEM((_SB,), jnp.int32),
            pltpu.VMEM((_SB,), jnp.int32),
            pltpu.VMEM((_SB,), jnp.int32),
            pltpu.VMEM((_SB,), jnp.int32),
            pltpu.VMEM((_SB,), jnp.int32),
            pltpu.VMEM((_SB,), jnp.int32),
            pltpu.VMEM((_SB,), jnp.int32),
            pltpu.VMEM((_SB,), jnp.int32),
            pltpu.VMEM((_SB, _CC), jnp.float32),
            pltpu.VMEM((_SB, _CC), jnp.float32),
            pltpu.VMEM((_SB, _CC), jnp.float32),
            pltpu.VMEM((_SB, _CC), jnp.float32),
            pltpu.VMEM_SHARED((_NPAD, _CC), jnp.float32),
            pltpu.SemaphoreType.DMA,
            pltpu.SemaphoreType.DMA,
            pltpu.SemaphoreType.DMA,
            pltpu.SemaphoreType.DMA,
            pltpu.SemaphoreType.DMA,
            pltpu.SemaphoreType.DMA,
            pltpu.SemaphoreType.DMA,
            pltpu.SemaphoreType.DMA,
        ],
    )(_sc_seg_body)
    return f(src3t, dst3, sigma2, vhflat, zeros_nc)


def _node1_body(uh_ref, seg_ref, t_ref, stats_ref):
    i = pl.program_id(0)
    num = jnp.concatenate([seg_ref[0], seg_ref[1]], axis=1)
    den = jnp.concatenate([seg_ref[2], seg_ref[3]], axis=1)
    t = uh_ref[...] + num / (den + EPS)
    t_ref[...] = t

    @pl.when(i == 0)
    def _():
        stats_ref[...] = jnp.zeros_like(stats_ref)

    stats_ref[0:1, :] += jnp.sum(t, axis=0, keepdims=True)
    stats_ref[1:2, :] += jnp.sum(t * t, axis=0, keepdims=True)


def _node1(uh, seg):
    grid = (N // _NB,)
    return pl.pallas_call(
        _node1_body,
        grid=grid,
        in_specs=[
            pl.BlockSpec((_NB, D), lambda i: (i, 0)),
            pl.BlockSpec((4, _NB, _CC), lambda i: (0, i, 0)),
        ],
        out_specs=[
            pl.BlockSpec((_NB, D), lambda i: (i, 0)),
            pl.BlockSpec((8, D), lambda i: (0, 0)),
        ],
        out_shape=[
            jax.ShapeDtypeStruct((N, D), jnp.float32),
            jax.ShapeDtypeStruct((8, D), jnp.float32),
        ],
    )(uh, seg)


def _node2_body(t_ref, h0_ref, stats_ref, gh_ref, bh_ref, out_ref):
    mean = stats_ref[0:1, :] * (1.0 / N)
    var = stats_ref[1:2, :] * (1.0 / N) - mean * mean
    rstd = lax.rsqrt(var + EPS)
    norm = (t_ref[...] - mean) * (rstd * gh_ref[...]) + bh_ref[...]
    out_ref[...] = h0_ref[...] + jnp.maximum(norm, 0.0)


def _node2(t, h0, stats, gamma_h, beta_h):
    grid = (N // _NB,)
    return pl.pallas_call(
        _node2_body,
        grid=grid,
        in_specs=[
            pl.BlockSpec((_NB, D), lambda i: (i, 0)),
            pl.BlockSpec((_NB, D), lambda i: (i, 0)),
            pl.BlockSpec((8, D), lambda i: (0, 0)),
            pl.BlockSpec((1, D), lambda i: (0, 0)),
            pl.BlockSpec((1, D), lambda i: (0, 0)),
        ],
        out_specs=pl.BlockSpec((_NB, D), lambda i: (i, 0)),
        out_shape=jax.ShapeDtypeStruct((N, D), jnp.float32),
    )(t, h0, stats, gamma_h, beta_h)


def kernel(h, e, edge_index, W_emb_n, W_emb_e, W_eta, WU, bU, WV, bV,
           W1, b1, W2, b2, W3, b3, gamma_h, beta_h, gamma_e, beta_e):
    src = edge_index[0]
    dst = edge_index[1]
    src3 = src.reshape(_GW, _GNB, _GB)
    dst3 = dst.reshape(_GW, _GNB, _GB)
    srct = src.reshape(16, _SNB, _SB)
    dstt = dst.reshape(16, _SNB, _SB)

    Wstack = jnp.stack([W_emb_n, WU, WV, W2, W3])
    bstack = jnp.stack([jnp.zeros_like(bU), bU, bV, b2, b3])[:, None, :]
    HW = _node_mm(h, Wstack, bstack)
    h0, Uh, Vh, W2h, W3h = HW[0], HW[1], HW[2], HW[3], HW[4]

    g = _sc_gather(src3, dst3, W2h, W3h)

    stats_e = _edge1(e, g, W1, b1[None, :])
    sigma2 = _edge2a(e, g, stats_e, b1[None, :], gamma_e[None, :],
                     beta_e[None, :], W1, W_emb_e, W_eta)
    e_new = _edge2b(g, e, stats_e, b1[None, :], gamma_e[None, :],
                    beta_e[None, :], W1, W_emb_e)

    vhflat = jnp.concatenate([Vh[:, 0:_CC], Vh[:, _CC:2 * _CC]], axis=0)
    zeros_nc = jnp.zeros((_RPT, _CC), jnp.float32)
    seg = _sc_seg(srct, dstt, sigma2, vhflat, zeros_nc)

    t, stats_h = _node1(Uh, seg)
    h_new = _node2(t, h0, stats_h, gamma_h[None, :], beta_h[None, :])
    return (h_new, e_new)


# speedup vs baseline: 1.0398x; 1.0398x over previous
import functools

import jax
import jax.numpy as jnp
from jax import lax
from jax.experimental import pallas as pl
from jax.experimental.pallas import tpu as pltpu
from jax.experimental.pallas import tpu_sc as plsc

N = 10000
E = 160000
D = 256
EPS = 1e-5


_NB = 1000


def _node_mm_body(h_ref, w_ref, b_ref, out_ref):
    out_ref[0] = (
        jnp.dot(h_ref[...], w_ref[0], preferred_element_type=jnp.float32)
        + b_ref[0]
    )


def _node_mm(h, Wstack, bstack):
    grid = (N // _NB, 5)
    return pl.pallas_call(
        _node_mm_body,
        grid=grid,
        in_specs=[
            pl.BlockSpec((_NB, D), lambda i, j: (i, 0)),
            pl.BlockSpec((1, D, D), lambda i, j: (j, 0, 0)),
            pl.BlockSpec((1, 1, D), lambda i, j: (j, 0, 0)),
        ],
        out_specs=pl.BlockSpec((1, _NB, D), lambda i, j: (j, i, 0)),
        out_shape=jax.ShapeDtypeStruct((5, N, D), jnp.float32),
    )(h, Wstack, bstack)


_GW = 32
_EPT = E // _GW
_GB = 40
_GNB = _EPT // _GB


def _sc_gather_body(src3, dst3, w2h_hbm, w3h_hbm,
                    g_out,
                    srcb, dstb, b2a, b2b, b3a, b3b,
                    sidx, sg0, sg1, sw0, sw1):
    wid = lax.axis_index("s") * 2 + lax.axis_index("c")
    base = wid * _EPT
    b2 = [b2a, b2b]
    b3 = [b3a, b3b]
    sg = [sg0, sg1]
    sw = [sw0, sw1]

    pltpu.async_copy(src3.at[wid], srcb, sidx).wait()
    pltpu.async_copy(dst3.at[wid], dstb, sidx).wait()

    def fire_g(b, j):
        pltpu.async_copy(w2h_hbm.at[srcb.at[b]], b2[j], sg[j])
        pltpu.async_copy(w3h_hbm.at[dstb.at[b]], b3[j], sg[j])

    def wait_g(b, j):
        pltpu.make_async_copy(w2h_hbm.at[srcb.at[b]], b2[j], sg[j]).wait()
        pltpu.make_async_copy(w3h_hbm.at[dstb.at[b]], b3[j], sg[j]).wait()

    def fire_w(b, j):
        off = base + b * _GB
        pltpu.async_copy(b2[j], g_out.at[pl.ds(off, _GB)], sw[j])

    def wait_w(b, j):
        off = base + b * _GB
        pltpu.make_async_copy(b2[j], g_out.at[pl.ds(off, _GB)], sw[j]).wait()

    def compute(j):
        def row_body(r, carry):
            for cv in range(16):
                sl = pl.ds(cv * 16, 16)
                b2[j][r, sl] = b2[j][r, sl] + b3[j][r, sl]
            return carry
        lax.fori_loop(0, _GB, row_body, 0)

    fire_g(0, 0)

    def outer(r, carry):
        for jj in range(2):
            b = r * 2 + jj

            @pl.when(b < _GNB)
            def _():
                @pl.when(b >= 1)
                def _():
                    wait_w(b - 1, 1 - jj)

                @pl.when(b + 1 < _GNB)
                def _():
                    fire_g(b + 1, 1 - jj)

                wait_g(b, jj)
                compute(jj)
                fire_w(b, jj)
        return carry

    lax.fori_loop(0, (_GNB + 2) // 2, outer, 0)
    wait_w(_GNB - 1, (_GNB - 1) % 2)


def _sc_gather(src3, dst3, w2h, w3h):
    mesh = plsc.VectorSubcoreMesh(core_axis_name="c", subcore_axis_name="s")
    f = functools.partial(
        pl.kernel,
        out_type=jax.ShapeDtypeStruct((E, D), jnp.float32),
        mesh=mesh,
        scratch_types=[
            pltpu.VMEM((_GNB, _GB), jnp.int32),
            pltpu.VMEM((_GNB, _GB), jnp.int32),
            pltpu.VMEM((_GB, D), jnp.float32),
            pltpu.VMEM((_GB, D), jnp.float32),
            pltpu.VMEM((_GB, D), jnp.float32),
            pltpu.VMEM((_GB, D), jnp.float32),
            pltpu.SemaphoreType.DMA,
            pltpu.SemaphoreType.DMA,
            pltpu.SemaphoreType.DMA,
            pltpu.SemaphoreType.DMA,
            pltpu.SemaphoreType.DMA,
        ],
    )(_sc_gather_body)
    return f(src3, dst3, w2h, w3h)


_EB = 2000


def _edge1_body(e_ref, g_ref, w1_ref, b1_ref, stats_ref):
    i = pl.program_id(0)
    w1e = jnp.dot(e_ref[...], w1_ref[...], preferred_element_type=jnp.float32)
    pre = w1e + b1_ref[...] + g_ref[...]

    @pl.when(i == 0)
    def _():
        stats_ref[...] = jnp.zeros_like(stats_ref)

    stats_ref[0:1, :] += jnp.sum(pre, axis=0, keepdims=True)
    stats_ref[1:2, :] += jnp.sum(pre * pre, axis=0, keepdims=True)


def _edge1(e, g, W1, b1):
    grid = (E // _EB,)
    return pl.pallas_call(
        _edge1_body,
        grid=grid,
        in_specs=[
            pl.BlockSpec((_EB, 16), lambda i: (i, 0)),
            pl.BlockSpec((_EB, D), lambda i: (i, 0)),
            pl.BlockSpec((16, D), lambda i: (0, 0)),
            pl.BlockSpec((1, D), lambda i: (0, 0)),
        ],
        out_specs=pl.BlockSpec((8, D), lambda i: (0, 0)),
        out_shape=jax.ShapeDtypeStruct((8, D), jnp.float32),
    )(e, g, W1, b1)


def _enew_block(e_blk, g_blk, stats, b1, ge, be, w1, wemb):
    mean = stats[0:1, :] * (1.0 / E)
    var = stats[1:2, :] * (1.0 / E) - mean * mean
    rstd = lax.rsqrt(var + EPS)
    w1e = jnp.dot(e_blk, w1, preferred_element_type=jnp.float32)
    pre = w1e + b1 + g_blk
    norm = (pre - mean) * (rstd * ge) + be
    e0 = jnp.dot(e_blk, wemb, preferred_element_type=jnp.float32)
    return e0 + jnp.maximum(norm, 0.0)


def _edge2_body(g_ref, e_ref, stats_ref, b1_ref, ge_ref, be_ref,
                w1_ref, wemb_ref, weta_ref, enew_ref, pay_ref):
    e_new = _enew_block(e_ref[...], g_ref[...], stats_ref[...], b1_ref[...],
                        ge_ref[...], be_ref[...], w1_ref[...], wemb_ref[...])
    enew_ref[...] = e_new
    sig = jax.nn.sigmoid(e_new)
    sigma = jnp.dot(sig, weta_ref[...], preferred_element_type=jnp.float32)
    pay_ref[0] = sigma[:, 0:128]
    pay_ref[1] = sigma[:, 128:256]


def _edge2(g, e, stats, b1, gamma_e, beta_e, W1, W_emb_e, W_eta):
    grid = (E // _EB,)
    return pl.pallas_call(
        _edge2_body,
        grid=grid,
        in_specs=[
            pl.BlockSpec((_EB, D), lambda i: (i, 0)),
            pl.BlockSpec((_EB, 16), lambda i: (i, 0)),
            pl.BlockSpec((8, D), lambda i: (0, 0)),
            pl.BlockSpec((1, D), lambda i: (0, 0)),
            pl.BlockSpec((1, D), lambda i: (0, 0)),
            pl.BlockSpec((1, D), lambda i: (0, 0)),
            pl.BlockSpec((16, D), lambda i: (0, 0)),
            pl.BlockSpec((16, D), lambda i: (0, 0)),
            pl.BlockSpec((D, D), lambda i: (0, 0)),
        ],
        out_specs=[
            pl.BlockSpec((_EB, D), lambda i: (i, 0)),
            pl.BlockSpec((2, _EB, _CC), lambda i: (0, i, 0)),
        ],
        out_shape=[
            jax.ShapeDtypeStruct((E, D), jnp.float32),
            jax.ShapeDtypeStruct((2, E, _CC), jnp.float32),
        ],
        input_output_aliases={0: 0},
    )(g, e, stats, b1, gamma_e, beta_e, W1, W_emb_e, W_eta)


_SB = 80
_EPS_T = E // 16
_SNB = _EPS_T // _SB
_NPAD = 10112
_RPT = _NPAD // 16
_CC = 128


def _sc_seg_body(src3t, dst3, sig_hbm, vh_hbm, zeros_hbm, out_hbm,
                 isa, isb, ida, idb, i2a, i2b, isca, iscb, sa, sb, va, vb,
                 acc, si0, si1, sv0, sv1, sg0, sg1, ss0, ss1):
    core = lax.axis_index("c")
    tile = lax.axis_index("s")
    rbase = tile * _RPT
    cN = core * N
    ibs = [isa, isb]
    ibd = [ida, idb]
    i2 = [i2a, i2b]
    isc = [isca, iscb]
    sgb = [sa, sb]
    vhb = [va, vb]
    si = [si0, si1]
    sv = [sv0, sv1]
    sg = [sg0, sg1]
    ss = [ss0, ss1]

    for phase in range(2):
        pltpu.sync_copy(zeros_hbm, acc.at[pl.ds(rbase, _RPT)])
        plsc.subcore_barrier()

        def fire_idx(b, j):
            pltpu.async_copy(dst3.at[tile, b], ibd[j], si[j])
            if phase == 0:
                pltpu.async_copy(src3t.at[tile, b], ibs[j], si[j])

        def wait_idx(b, j):
            pltpu.make_async_copy(dst3.at[tile, b], ibd[j], si[j]).wait()
            if phase == 0:
                pltpu.make_async_copy(src3t.at[tile, b], ibs[j],
                                      si[j]).wait()

        def build(j):
            for q in range(_SB // 16):
                qs = pl.ds(q * 16, 16)
                isc[j][qs] = ibd[j][qs]
                if phase == 0:
                    i2[j][qs] = ibs[j][qs] + cN

        def fire_sig(b, j):
            off = tile * _EPS_T + b * _SB
            pltpu.async_copy(sig_hbm.at[core, pl.ds(off, _SB)],
                             sgb[j], sg[j])

        def wait_sig(b, j):
            off = tile * _EPS_T + b * _SB
            pltpu.make_async_copy(sig_hbm.at[core, pl.ds(off, _SB)],
                                  sgb[j], sg[j]).wait()

        def fire_vh(j):
            pltpu.async_copy(vh_hbm.at[i2[j]], vhb[j], sv[j])

        def wait_vh(j):
            pltpu.make_async_copy(vh_hbm.at[i2[j]], vhb[j], sv[j]).wait()

        def compute(j):
            def row_body(r, carry):
                for cv in range(_CC // 16):
                    slc = pl.ds(cv * 16, 16)
                    vhb[j][r, slc] = vhb[j][r, slc] * sgb[j][r, slc]
                return carry
            lax.fori_loop(0, _SB, row_body, 0)

        def fire_scat(j):
            buf = vhb[j] if phase == 0 else sgb[j]
            pltpu.async_copy(buf, acc.at[isc[j]], ss[j], add=True)

        def wait_scat(j):
            buf = vhb[j] if phase == 0 else sgb[j]
            pltpu.make_async_copy(buf, acc.at[isc[j]], ss[j]).wait()

        fire_idx(0, 0)
        fire_idx(1, 1)
        wait_idx(0, 0)
        build(0)
        if phase == 0:
            fire_vh(0)
        fire_sig(0, 0)

        def outer(r, carry):
            for jj in range(2):
                b = r * 2 + jj

                @pl.when(b < _SNB)
                def _():
                    @pl.when(b >= 1)
                    def _():
                        wait_scat(1 - jj)

                    @pl.when(b + 1 < _SNB)
                    def _():
                        wait_idx(b + 1, 1 - jj)
                        build(1 - jj)
                        if phase == 0:
                            fire_vh(1 - jj)
                        fire_sig(b + 1, 1 - jj)

                    @pl.when(b + 2 < _SNB)
                    def _():
                        fire_idx(b + 2, jj)

                    wait_sig(b, jj)
                    if phase == 0:
                        wait_vh(jj)
                        compute(jj)
                    fire_scat(jj)
            return carry

        lax.fori_loop(0, (_SNB + 2) // 2, outer, 0)
        wait_scat((_SNB - 1) % 2)
        plsc.subcore_barrier()
        pltpu.sync_copy(acc.at[pl.ds(rbase, _RPT)],
                        out_hbm.at[core + 2 * phase, pl.ds(rbase, _RPT)])


def _sc_seg(src3t, dst3, sigma2, vhflat, zeros_nc):
    mesh = plsc.VectorSubcoreMesh(core_axis_name="c", subcore_axis_name="s")
    f = functools.partial(
        pl.kernel,
        out_type=jax.ShapeDtypeStruct((4, _NPAD, _CC), jnp.float32),
        mesh=mesh,
        scratch_types=[
            pltpu.VMEM((_SB,), jnp.int32),
            pltpu.VMEM((_SB,), jnp.int32),
            pltpu.VMEM((_SB,), jnp.int32),
            pltpu.VMEM((_SB,), jnp.int32),
            pltpu.VMEM((_SB,), jnp.int32),
            pltpu.VMEM((_SB,), jnp.int32),
            pltpu.VMEM((_SB,), jnp.int32),
            pltpu.VMEM((_SB,), jnp.int32),
            pltpu.VMEM((_SB, _CC), jnp.float32),
            pltpu.VMEM((_SB, _CC), jnp.float32),
            pltpu.VMEM((_SB, _CC), jnp.float32),
            pltpu.VMEM((_SB, _CC), jnp.float32),
            pltpu.VMEM_SHARED((_NPAD, _CC), jnp.float32),
            pltpu.SemaphoreType.DMA,
            pltpu.SemaphoreType.DMA,
            pltpu.SemaphoreType.DMA,
            pltpu.SemaphoreType.DMA,
            pltpu.SemaphoreType.DMA,
            pltpu.SemaphoreType.DMA,
            pltpu.SemaphoreType.DMA,
            pltpu.SemaphoreType.DMA,
        ],
    )(_sc_seg_body)
    return f(src3t, dst3, sigma2, vhflat, zeros_nc)


def _node1_body(uh_ref, seg_ref, t_ref, stats_ref):
    i = pl.program_id(0)
    num = jnp.concatenate([seg_ref[0], seg_ref[1]], axis=1)
    den = jnp.concatenate([seg_ref[2], seg_ref[3]], axis=1)
    t = uh_ref[...] + num / (den + EPS)
    t_ref[...] = t

    @pl.when(i == 0)
    def _():
        stats_ref[...] = jnp.zeros_like(stats_ref)

    stats_ref[0:1, :] += jnp.sum(t, axis=0, keepdims=True)
    stats_ref[1:2, :] += jnp.sum(t * t, axis=0, keepdims=True)


def _node1(uh, seg):
    grid = (N // _NB,)
    return pl.pallas_call(
        _node1_body,
        grid=grid,
        in_specs=[
            pl.BlockSpec((_NB, D), lambda i: (i, 0)),
            pl.BlockSpec((4, _NB, _CC), lambda i: (0, i, 0)),
        ],
        out_specs=[
            pl.BlockSpec((_NB, D), lambda i: (i, 0)),
            pl.BlockSpec((8, D), lambda i: (0, 0)),
        ],
        out_shape=[
            jax.ShapeDtypeStruct((N, D), jnp.float32),
            jax.ShapeDtypeStruct((8, D), jnp.float32),
        ],
    )(uh, seg)


def _node2_body(t_ref, h0_ref, stats_ref, gh_ref, bh_ref, out_ref):
    mean = stats_ref[0:1, :] * (1.0 / N)
    var = stats_ref[1:2, :] * (1.0 / N) - mean * mean
    rstd = lax.rsqrt(var + EPS)
    norm = (t_ref[...] - mean) * (rstd * gh_ref[...]) + bh_ref[...]
    out_ref[...] = h0_ref[...] + jnp.maximum(norm, 0.0)


def _node2(t, h0, stats, gamma_h, beta_h):
    grid = (N // _NB,)
    return pl.pallas_call(
        _node2_body,
        grid=grid,
        in_specs=[
            pl.BlockSpec((_NB, D), lambda i: (i, 0)),
            pl.BlockSpec((_NB, D), lambda i: (i, 0)),
            pl.BlockSpec((8, D), lambda i: (0, 0)),
            pl.BlockSpec((1, D), lambda i: (0, 0)),
            pl.BlockSpec((1, D), lambda i: (0, 0)),
        ],
        out_specs=pl.BlockSpec((_NB, D), lambda i: (i, 0)),
        out_shape=jax.ShapeDtypeStruct((N, D), jnp.float32),
    )(t, h0, stats, gamma_h, beta_h)


def kernel(h, e, edge_index, W_emb_n, W_emb_e, W_eta, WU, bU, WV, bV,
           W1, b1, W2, b2, W3, b3, gamma_h, beta_h, gamma_e, beta_e):
    src = edge_index[0]
    dst = edge_index[1]
    src3 = src.reshape(_GW, _GNB, _GB)
    dst3 = dst.reshape(_GW, _GNB, _GB)
    srct = src.reshape(16, _SNB, _SB)
    dstt = dst.reshape(16, _SNB, _SB)

    Wstack = jnp.stack([W_emb_n, WU, WV, W2, W3])
    bstack = jnp.stack([jnp.zeros_like(bU), bU, bV, b2, b3])[:, None, :]
    HW = _node_mm(h, Wstack, bstack)
    h0, Uh, Vh, W2h, W3h = HW[0], HW[1], HW[2], HW[3], HW[4]

    g = _sc_gather(src3, dst3, W2h, W3h)

    stats_e = _edge1(e, g, W1, b1[None, :])
    e_new, sigma2 = _edge2(g, e, stats_e, b1[None, :], gamma_e[None, :],
                           beta_e[None, :], W1, W_emb_e, W_eta)

    vhflat = jnp.concatenate([Vh[:, 0:_CC], Vh[:, _CC:2 * _CC]], axis=0)
    zeros_nc = jnp.zeros((_RPT, _CC), jnp.float32)
    seg = _sc_seg(srct, dstt, sigma2, vhflat, zeros_nc)

    t, stats_h = _node1(Uh, seg)
    h_new = _node2(t, h0, stats_h, gamma_h[None, :], beta_h[None, :])
    return (h_new, e_new)
